# baseline (speedup 1.0000x reference)
import functools

import jax
import jax.numpy as jnp
from jax import lax
from jax.experimental import pallas as pl
from jax.experimental.pallas import tpu as pltpu
from jax.experimental.pallas import tpu_sc as plsc

SP_LEN = 2048
EMBED_DIM = 4096

NUM_CORES = 2
NUM_SUBCORES = 16
NUM_WORKERS = NUM_CORES * NUM_SUBCORES
ROWS_PER_WORKER = SP_LEN // NUM_WORKERS
CHUNK = 8
NUM_CHUNKS = ROWS_PER_WORKER // CHUNK
NBUF = 2
OUTER = NUM_CHUNKS // NBUF


def _gather_body(table_hbm, idx_hbm, out_hbm, idx_v, rows_v, gsem, ssem):
    wid = lax.axis_index("s") * NUM_CORES + lax.axis_index("c")
    base = wid * ROWS_PER_WORKER

    pltpu.sync_copy(idx_hbm.at[pl.ds(base, ROWS_PER_WORKER)], idx_v)

    def start_gather(c, b):
        pltpu.async_copy(
            table_hbm.at[idx_v.at[pl.ds(c * CHUNK, CHUNK)]],
            rows_v.at[b],
            gsem.at[b],
        )

    def wait_gather(b):
        pltpu.make_async_copy(
            table_hbm.at[pl.ds(0, CHUNK)], rows_v.at[b], gsem.at[b]
        ).wait()

    def start_store(c, b):
        pltpu.async_copy(
            rows_v.at[b], out_hbm.at[pl.ds(base + c * CHUNK, CHUNK)], ssem.at[b]
        )

    def wait_store(b):
        pltpu.make_async_copy(
            rows_v.at[b], out_hbm.at[pl.ds(base, CHUNK)], ssem.at[b]
        ).wait()

    for b in range(NBUF):
        start_gather(b, b)

    def outer(o, carry):
        for b in range(NBUF):
            c = o * NBUF + b
            wait_gather(b)
            start_store(c, b)
            nxt = c + NBUF

            @pl.when(nxt < NUM_CHUNKS)
            def _():
                wait_store(b)
                start_gather(nxt, b)

        return carry

    lax.fori_loop(0, OUTER, outer, 0)

    for b in range(NBUF):
        wait_store(b)


@jax.jit
def _soft_prompt_lookup(soft_prompt, seq_indices):
    mesh = plsc.VectorSubcoreMesh(core_axis_name="c", subcore_axis_name="s")
    run = functools.partial(
        pl.kernel,
        mesh=mesh,
        out_type=jax.ShapeDtypeStruct((SP_LEN, EMBED_DIM), jnp.float32),
        scratch_types=[
            pltpu.VMEM((ROWS_PER_WORKER,), jnp.int32),
            pltpu.VMEM((NBUF, CHUNK, EMBED_DIM), jnp.float32),
            pltpu.SemaphoreType.DMA((NBUF,)),
            pltpu.SemaphoreType.DMA((NBUF,)),
        ],
    )(_gather_body)
    return run(soft_prompt, seq_indices)


def kernel(soft_prompt, seq_indices):
    return _soft_prompt_lookup(soft_prompt, seq_indices.astype(jnp.int32))
